# baseline (speedup 1.0000x reference)
import functools

import jax
import jax.numpy as jnp
from jax import lax
from jax.experimental import pallas as pl
from jax.experimental.pallas import tpu as pltpu
from jax.experimental.pallas import tpu_sc as plsc

B, F, D, T = 4, 2048, 128, 256
K = 15
NW = 32
TPW = (B * T) // NW
LANES = 16
CPT = D // LANES
STEP = F // T
SPAN = 272
HALF0 = 152
HALF1 = SPAN - HALF0

_mesh = plsc.VectorSubcoreMesh(core_axis_name="c", subcore_axis_name="s")


@functools.partial(
    pl.kernel,
    mesh=_mesh,
    out_type=jax.ShapeDtypeStruct((B * T, D), jnp.float32),
    scratch_types=[
        pltpu.VMEM((TPW,), jnp.int32),
        pltpu.VMEM((SPAN, D), jnp.float32),
        pltpu.VMEM((TPW, D), jnp.float32),
        pltpu.SemaphoreType.DMA,
        pltpu.SemaphoreType.DMA,
    ],
    compiler_params=pltpu.CompilerParams(
        use_tc_tiling_on_sc=False, needs_layout_passes=False
    ),
)
def _tatum_pool_sc(feat_hbm, packed_hbm, out_hbm, pv, rowsv, outv, sem0, sem1):
    c = lax.axis_index("c")
    s = lax.axis_index("s")
    w = c * 16 + s
    b = w // (T // TPW)
    t0 = (w % (T // TPW)) * TPW

    base_row = jnp.minimum(b * F + STEP * t0, B * F - SPAN)
    off0 = base_row - b * F

    cp0 = pltpu.async_copy(feat_hbm.at[pl.ds(base_row, HALF0)],
                           rowsv.at[pl.ds(0, HALF0)], sem0)
    cp1 = pltpu.async_copy(feat_hbm.at[pl.ds(base_row + HALF0, HALF1)],
                           rowsv.at[pl.ds(HALF0, HALF1)], sem1)

    pltpu.sync_copy(packed_hbm.at[b, pl.ds(t0, TPW)], pv)

    def one(t):
        pk = jnp.max(plsc.load_gather(pv, [jnp.full((LANES,), t, jnp.int32)]))
        st = (pk & 0xFFFF) - off0
        em = (pk >> 16) - (off0 + 1)
        rk = [jnp.minimum(st + k, em) for k in range(K)]
        for cc in range(CPT):
            acc = rowsv[rk[0], pl.ds(cc * LANES, LANES)]
            for k in range(1, K):
                acc = jnp.maximum(acc, rowsv[rk[k], pl.ds(cc * LANES, LANES)])
            outv[t, pl.ds(cc * LANES, LANES)] = acc

    def body(i, carry):
        one(i * 2)
        one(i * 2 + 1)
        return carry

    cp0.wait()
    lax.fori_loop(0, TPW // 4, body, 0)
    cp1.wait()
    lax.fori_loop(TPW // 4, TPW // 2, body, 0)

    pltpu.sync_copy(outv, out_hbm.at[pl.ds(w * TPW, TPW)])


def kernel(featureMaps, tatumsBoundaries):
    feat2d = featureMaps.reshape(B * F, D)
    bnd = tatumsBoundaries.astype(jnp.int32)
    packed = bnd[..., 0] + (bnd[..., 1] << 16)
    out = _tatum_pool_sc(feat2d, packed)
    return out.reshape(B, T, D)


# speedup vs baseline: 1.0349x; 1.0349x over previous
import functools

import jax
import jax.numpy as jnp
from jax import lax
from jax.experimental import pallas as pl
from jax.experimental.pallas import tpu as pltpu
from jax.experimental.pallas import tpu_sc as plsc

B, F, D, T = 4, 2048, 128, 256
K = 15
NW = 32
TPW = (B * T) // NW
LANES = 16
CPT = D // LANES
STEP = F // T
SPAN = 272
HALF0 = 152
HALF1 = SPAN - HALF0

_mesh = plsc.VectorSubcoreMesh(core_axis_name="c", subcore_axis_name="s")


@functools.partial(
    pl.kernel,
    mesh=_mesh,
    out_type=jax.ShapeDtypeStruct((B * T, D), jnp.float32),
    scratch_types=[
        pltpu.VMEM((TPW,), jnp.int32),
        pltpu.VMEM((SPAN, D), jnp.float32),
        pltpu.VMEM((TPW, D), jnp.float32),
        pltpu.SemaphoreType.DMA,
        pltpu.SemaphoreType.DMA,
    ],
    compiler_params=pltpu.CompilerParams(
        use_tc_tiling_on_sc=False, needs_layout_passes=False
    ),
)
def _tatum_pool_sc(feat_hbm, packed_hbm, out_hbm, pv, rowsv, outv, sem0, sem1):
    c = lax.axis_index("c")
    s = lax.axis_index("s")
    w = c * 16 + s
    b = w // (T // TPW)
    t0 = (w % (T // TPW)) * TPW

    base_row = jnp.minimum(b * F + STEP * t0, B * F - SPAN)
    off0 = base_row - b * F

    cp0 = pltpu.async_copy(feat_hbm.at[pl.ds(base_row, HALF0)],
                           rowsv.at[pl.ds(0, HALF0)], sem0)
    cp1 = pltpu.async_copy(feat_hbm.at[pl.ds(base_row + HALF0, HALF1)],
                           rowsv.at[pl.ds(HALF0, HALF1)], sem1)

    pltpu.sync_copy(packed_hbm.at[b, pl.ds(t0, TPW)], pv)

    def reduce_k(t, st, em, kmax):
        rk = [jnp.minimum(st + k, em) for k in range(kmax)]
        for cc in range(CPT):
            acc = rowsv[rk[0], pl.ds(cc * LANES, LANES)]
            for k in range(1, kmax):
                acc = jnp.maximum(acc, rowsv[rk[k], pl.ds(cc * LANES, LANES)])
            outv[t, pl.ds(cc * LANES, LANES)] = acc

    def body(t, carry):
        pk = jnp.max(plsc.load_gather(pv, [jnp.full((LANES,), t, jnp.int32)]))
        st = (pk & 0xFFFF) - off0
        em = (pk >> 16) - (off0 + 1)
        lax.cond(em - st < STEP,
                 lambda: reduce_k(t, st, em, STEP),
                 lambda: reduce_k(t, st, em, K))
        return carry

    cp0.wait()
    lax.fori_loop(0, TPW // 2, body, 0)
    cp1.wait()
    lax.fori_loop(TPW // 2, TPW, body, 0)

    pltpu.sync_copy(outv, out_hbm.at[pl.ds(w * TPW, TPW)])


def kernel(featureMaps, tatumsBoundaries):
    feat2d = featureMaps.reshape(B * F, D)
    bnd = tatumsBoundaries.astype(jnp.int32)
    packed = bnd[..., 0] + (bnd[..., 1] << 16)
    out = _tatum_pool_sc(feat2d, packed)
    return out.reshape(B, T, D)
